# baseline (speedup 1.0000x reference)
import functools
import math

import numpy as np
import jax
import jax.numpy as jnp
from jax import lax
from jax.experimental import pallas as pl
from jax.experimental.pallas import tpu as pltpu
from jax.experimental.pallas import tpu_sc as plsc

_B = 16384
_D = 64
_N = 1000000

_info = plsc.get_sparse_core_info()
_NC, _NS, _L = _info.num_cores, _info.num_subcores, _info.num_lanes
_NW = _NC * _NS
_B_PER_W = _B // _NW

_CROWS = 256
_NCHUNKS = (_N + _CROWS - 1) // _CROWS
_PAIR_STEPS = ((_NCHUNKS + _NW - 1) // _NW + 1) // 2

_CHUNK = 64


def _tf2x32(k1, k2, x0, x1):
    ks0 = np.uint32(k1)
    ks1 = np.uint32(k2)
    ks2 = np.uint32(ks0 ^ ks1 ^ np.uint32(0x1BD11BDA))
    x0 = x0.astype(np.uint32)
    x1 = x1.astype(np.uint32)
    rot0 = (13, 15, 26, 6)
    rot1 = (17, 29, 16, 24)

    def rnd(x0, x1, rots):
        for r in rots:
            x0 = (x0 + x1).astype(np.uint32)
            x1 = ((x1 << np.uint32(r)) | (x1 >> np.uint32(32 - r))).astype(np.uint32)
            x1 = x0 ^ x1
        return x0, x1

    x0 = (x0 + ks0).astype(np.uint32)
    x1 = (x1 + ks1).astype(np.uint32)
    x0, x1 = rnd(x0, x1, rot0)
    x0 = (x0 + ks1).astype(np.uint32); x1 = (x1 + ks2 + np.uint32(1)).astype(np.uint32)
    x0, x1 = rnd(x0, x1, rot1)
    x0 = (x0 + ks2).astype(np.uint32); x1 = (x1 + ks0 + np.uint32(2)).astype(np.uint32)
    x0, x1 = rnd(x0, x1, rot0)
    x0 = (x0 + ks0).astype(np.uint32); x1 = (x1 + ks1 + np.uint32(3)).astype(np.uint32)
    x0, x1 = rnd(x0, x1, rot1)
    x0 = (x0 + ks1).astype(np.uint32); x1 = (x1 + ks2 + np.uint32(4)).astype(np.uint32)
    x0, x1 = rnd(x0, x1, rot0)
    x0 = (x0 + ks2).astype(np.uint32); x1 = (x1 + ks0 + np.uint32(5)).astype(np.uint32)
    return x0, x1


def _erfinv_f32(x):
    x = x.astype(np.float32)
    w = -np.log1p((-x * x).astype(np.float32)).astype(np.float32)
    small = w < np.float32(5.0)
    ws = (w - np.float32(2.5)).astype(np.float32)
    wl = (np.sqrt(w.astype(np.float32)) - np.float32(3.0)).astype(np.float32)
    cs = (2.81022636e-08, 3.43273939e-07, -3.5233877e-06, -4.39150654e-06,
          0.00021858087, -0.00125372503, -0.00417768164, 0.246640727, 1.50140941)
    cl = (-0.000200214257, 0.000100950558, 0.00134934322, -0.00367342844,
          0.00573950773, -0.0076224613, 0.00943887047, 1.00167406, 2.83297682)
    ps = np.float32(cs[0])
    for c in cs[1:]:
        ps = (np.float32(c) + ps * ws).astype(np.float32)
    plg = np.float32(cl[0])
    for c in cl[1:]:
        plg = (np.float32(c) + plg * wl).astype(np.float32)
    return (np.where(small, ps, plg) * x).astype(np.float32)


@functools.lru_cache(maxsize=None)
def _eps_consts():
    k1, k2 = np.uint32(0), np.uint32(42)
    b1, b2 = _tf2x32(k1, k2, np.array([0, 0], np.uint32),
                     np.array([0, 1], np.uint32))
    n = _B * _D
    hi = np.zeros(n, np.uint32)
    lo = np.arange(n, dtype=np.uint32)
    out = []
    for kk1, kk2 in ((b1[0], b2[0]), (b1[1], b2[1])):
        r1, r2 = _tf2x32(kk1, kk2, hi, lo)
        bits = (r1 ^ r2).astype(np.uint32)
        float_bits = (bits >> np.uint32(9)) | np.uint32(0x3F800000)
        floats = float_bits.view(np.float32) - np.float32(1.0)
        minval = np.nextafter(np.float32(-1.0), np.float32(0.0), dtype=np.float32)
        u = np.maximum(minval, (floats * (np.float32(1.0) - minval)
                                + minval).astype(np.float32))
        out.append((np.float32(math.sqrt(2)) * _erfinv_f32(u))
                   .astype(np.float32).reshape(_B, _D))
    return out[0], out[1]


def _relayout_body(t_mu_s, t_lv_s, t_mu_a, t_lv_a,
                   s_mu_s, s_lv_s, s_mu_a, s_lv_a,
                   buf0, buf1, isem0, isem1, osem0, osem1):
    wid = lax.axis_index("s") * _NC + lax.axis_index("c")
    bufs = ((buf0, isem0, osem0), (buf1, isem1, osem1))

    for tbl, scr in ((t_mu_s, s_mu_s), (t_lv_s, s_lv_s),
                     (t_mu_a, s_mu_a), (t_lv_a, s_lv_a)):

        def pair_fn(p, carry, tbl=tbl, scr=scr):
            for b, (buf, isem, osem) in enumerate(bufs):
                s = 2 * p + b
                c = wid + _NW * s

                @pl.when(c < _NCHUNKS)
                def _(buf=buf, isem=isem, osem=osem, s=s, c=c):
                    @pl.when(s >= 2)
                    def _():
                        pltpu.make_async_copy(
                            buf.reshape(_CROWS // 2, 2 * _D),
                            scr.at[pl.ds(0, _CROWS // 2)], osem).wait()
                    r = jnp.minimum(c * _CROWS, _N - _CROWS)
                    pltpu.async_copy(tbl.at[pl.ds(r, _CROWS)], buf, isem)
            for b, (buf, isem, osem) in enumerate(bufs):
                s = 2 * p + b
                c = wid + _NW * s

                @pl.when(c < _NCHUNKS)
                def _(buf=buf, isem=isem, osem=osem, c=c):
                    pltpu.make_async_copy(
                        tbl.at[pl.ds(0, _CROWS)], buf, isem).wait()
                    r = jnp.minimum(c * _CROWS, _N - _CROWS)
                    pltpu.async_copy(
                        buf.reshape(_CROWS // 2, 2 * _D),
                        scr.at[pl.ds(r // 2, _CROWS // 2)], osem)
            return carry

        lax.fori_loop(0, _PAIR_STEPS, pair_fn, 0)
        for buf, isem, osem in bufs:
            pltpu.make_async_copy(
                buf.reshape(_CROWS // 2, 2 * _D),
                scr.at[pl.ds(0, _CROWS // 2)], osem).wait()


def _gather_body(ids, s_mu_s, s_lv_s, s_mu_a, s_lv_a, eps_s_h, eps_a_h,
                 lat_s_o, lat_a_o, mu_s_o, lv_s_o, mu_a_o, lv_a_o,
                 idx_v, q_v, g_mu_s, g_lv_s, g_mu_a, g_lv_a,
                 eps_s_v, eps_a_v, o_mu_s, o_lv_s, o_mu_a, o_lv_a,
                 sem, esem):
    wid = lax.axis_index("s") * _NC + lax.axis_index("c")
    base0 = wid * _B_PER_W

    def chunk_fn(ci, carry0):
        base = base0 + ci * _CHUNK
        pltpu.sync_copy(ids.at[pl.ds(base, _CHUNK)], idx_v)
        e1 = pltpu.async_copy(eps_s_h.at[pl.ds(base, _CHUNK)], eps_s_v, esem)
        e2 = pltpu.async_copy(eps_a_h.at[pl.ds(base, _CHUNK)], eps_a_v, esem)

        def qidx_fn(g, carry):
            sl = pl.ds(g * _L, _L)
            q_v[sl] = lax.shift_right_logical(idx_v[sl], 1)
            return carry

        lax.fori_loop(0, _CHUNK // _L, qidx_fn, 0)

        copies = [
            pltpu.async_copy(s_mu_s.at[q_v], g_mu_s, sem),
            pltpu.async_copy(s_lv_s.at[q_v], g_lv_s, sem),
            pltpu.async_copy(s_mu_a.at[q_v], g_mu_a, sem),
            pltpu.async_copy(s_lv_a.at[q_v], g_lv_a, sem),
        ]
        for cp in copies:
            cp.wait()
        e1.wait()
        e2.wait()

        def row_fn(g, carry):
            hv = lax.bitwise_and(idx_v[pl.ds(g * _L, _L)], 1) * _D
            for k in range(_L):
                r = g * _L + k
                h = hv[k]
                for j in range(_D // _L):
                    sl = pl.ds(j * _L, _L)
                    hsl = pl.ds(h + j * _L, _L)
                    mu1 = g_mu_s[r, hsl]
                    lv1 = g_lv_s[r, hsl]
                    mu2 = g_mu_a[r, hsl]
                    lv2 = g_lv_a[r, hsl]
                    o_mu_s[r, sl] = mu1
                    o_lv_s[r, sl] = lv1
                    o_mu_a[r, sl] = mu2
                    o_lv_a[r, sl] = lv2
                    eps_s_v[r, sl] = mu1 + eps_s_v[r, sl] * jnp.exp(0.5 * lv1)
                    eps_a_v[r, sl] = mu2 + eps_a_v[r, sl] * jnp.exp(0.5 * lv2)
            return carry

        lax.fori_loop(0, _CHUNK // _L, row_fn, 0)

        dst = pl.ds(base, _CHUNK)
        pltpu.sync_copy(eps_s_v, lat_s_o.at[dst])
        pltpu.sync_copy(eps_a_v, lat_a_o.at[dst])
        pltpu.sync_copy(o_mu_s, mu_s_o.at[dst])
        pltpu.sync_copy(o_lv_s, lv_s_o.at[dst])
        pltpu.sync_copy(o_mu_a, mu_a_o.at[dst])
        pltpu.sync_copy(o_lv_a, lv_a_o.at[dst])
        return carry0

    lax.fori_loop(0, _B_PER_W // _CHUNK, chunk_fn, 0)


@functools.lru_cache(maxsize=None)
def _build_relayout():
    scr = jax.ShapeDtypeStruct((_N // 2, 2 * _D), jnp.float32)
    return pl.kernel(
        _relayout_body,
        mesh=plsc.VectorSubcoreMesh(core_axis_name="c", subcore_axis_name="s"),
        compiler_params=pltpu.CompilerParams(use_tc_tiling_on_sc=True),
        out_type=[scr] * 4,
        scratch_types=[
            pltpu.VMEM((_CROWS, _D), jnp.float32),
            pltpu.VMEM((_CROWS, _D), jnp.float32),
            pltpu.SemaphoreType.DMA,
            pltpu.SemaphoreType.DMA,
            pltpu.SemaphoreType.DMA,
            pltpu.SemaphoreType.DMA,
        ],
    )


@functools.lru_cache(maxsize=None)
def _build_gather():
    out = jax.ShapeDtypeStruct((_B, _D), jnp.float32)
    return pl.kernel(
        _gather_body,
        mesh=plsc.VectorSubcoreMesh(core_axis_name="c", subcore_axis_name="s"),
        compiler_params=pltpu.CompilerParams(use_tc_tiling_on_sc=True),
        out_type=[out] * 6,
        scratch_types=[
            pltpu.VMEM((_CHUNK,), jnp.int32),
            pltpu.VMEM((_CHUNK,), jnp.int32),
            pltpu.VMEM((_CHUNK, 2 * _D), jnp.float32),
            pltpu.VMEM((_CHUNK, 2 * _D), jnp.float32),
            pltpu.VMEM((_CHUNK, 2 * _D), jnp.float32),
            pltpu.VMEM((_CHUNK, 2 * _D), jnp.float32),
            pltpu.VMEM((_CHUNK, _D), jnp.float32),
            pltpu.VMEM((_CHUNK, _D), jnp.float32),
            pltpu.VMEM((_CHUNK, _D), jnp.float32),
            pltpu.VMEM((_CHUNK, _D), jnp.float32),
            pltpu.VMEM((_CHUNK, _D), jnp.float32),
            pltpu.VMEM((_CHUNK, _D), jnp.float32),
            pltpu.SemaphoreType.DMA,
            pltpu.SemaphoreType.DMA,
        ],
    )


def kernel(instance_ids, weight_mu_shape, weight_logvar_shape,
           weight_mu_app, weight_logvar_app):
    ids = instance_ids.astype(jnp.int32)
    eps_s, eps_a = _eps_consts()
    scr = [jnp.reshape(w, (_N // 2, 2 * _D))
           for w in (weight_mu_shape, weight_logvar_shape,
                     weight_mu_app, weight_logvar_app)]
    lat_s, lat_a, mu_s, lv_s, mu_a, lv_a = _build_gather()(
        ids, *scr, jnp.asarray(eps_s), jnp.asarray(eps_a))
    return (lat_s, lat_a, mu_s, lv_s, mu_a, lv_a)


# speedup vs baseline: 1.0105x; 1.0105x over previous
import functools
import math

import numpy as np
import jax
import jax.numpy as jnp
from jax import lax
from jax.experimental import pallas as pl
from jax.experimental.pallas import tpu as pltpu
from jax.experimental.pallas import tpu_sc as plsc

_B = 16384
_D = 64
_N = 1000000

_info = plsc.get_sparse_core_info()
_NC, _NS, _L = _info.num_cores, _info.num_subcores, _info.num_lanes
_NW = _NC * _NS
_B_PER_W = _B // _NW

_CROWS = 256
_NCHUNKS = (_N + _CROWS - 1) // _CROWS
_PAIR_STEPS = ((_NCHUNKS + _NW - 1) // _NW + 1) // 2

_CHUNK = 64


def _tf2x32(k1, k2, x0, x1):
    ks0 = np.uint32(k1)
    ks1 = np.uint32(k2)
    ks2 = np.uint32(ks0 ^ ks1 ^ np.uint32(0x1BD11BDA))
    x0 = x0.astype(np.uint32)
    x1 = x1.astype(np.uint32)
    rot0 = (13, 15, 26, 6)
    rot1 = (17, 29, 16, 24)

    def rnd(x0, x1, rots):
        for r in rots:
            x0 = (x0 + x1).astype(np.uint32)
            x1 = ((x1 << np.uint32(r)) | (x1 >> np.uint32(32 - r))).astype(np.uint32)
            x1 = x0 ^ x1
        return x0, x1

    x0 = (x0 + ks0).astype(np.uint32)
    x1 = (x1 + ks1).astype(np.uint32)
    x0, x1 = rnd(x0, x1, rot0)
    x0 = (x0 + ks1).astype(np.uint32); x1 = (x1 + ks2 + np.uint32(1)).astype(np.uint32)
    x0, x1 = rnd(x0, x1, rot1)
    x0 = (x0 + ks2).astype(np.uint32); x1 = (x1 + ks0 + np.uint32(2)).astype(np.uint32)
    x0, x1 = rnd(x0, x1, rot0)
    x0 = (x0 + ks0).astype(np.uint32); x1 = (x1 + ks1 + np.uint32(3)).astype(np.uint32)
    x0, x1 = rnd(x0, x1, rot1)
    x0 = (x0 + ks1).astype(np.uint32); x1 = (x1 + ks2 + np.uint32(4)).astype(np.uint32)
    x0, x1 = rnd(x0, x1, rot0)
    x0 = (x0 + ks2).astype(np.uint32); x1 = (x1 + ks0 + np.uint32(5)).astype(np.uint32)
    return x0, x1


def _erfinv_f32(x):
    x = x.astype(np.float32)
    w = -np.log1p((-x * x).astype(np.float32)).astype(np.float32)
    small = w < np.float32(5.0)
    ws = (w - np.float32(2.5)).astype(np.float32)
    wl = (np.sqrt(w.astype(np.float32)) - np.float32(3.0)).astype(np.float32)
    cs = (2.81022636e-08, 3.43273939e-07, -3.5233877e-06, -4.39150654e-06,
          0.00021858087, -0.00125372503, -0.00417768164, 0.246640727, 1.50140941)
    cl = (-0.000200214257, 0.000100950558, 0.00134934322, -0.00367342844,
          0.00573950773, -0.0076224613, 0.00943887047, 1.00167406, 2.83297682)
    ps = np.float32(cs[0])
    for c in cs[1:]:
        ps = (np.float32(c) + ps * ws).astype(np.float32)
    plg = np.float32(cl[0])
    for c in cl[1:]:
        plg = (np.float32(c) + plg * wl).astype(np.float32)
    return (np.where(small, ps, plg) * x).astype(np.float32)


@functools.lru_cache(maxsize=None)
def _eps_consts():
    k1, k2 = np.uint32(0), np.uint32(42)
    b1, b2 = _tf2x32(k1, k2, np.array([0, 0], np.uint32),
                     np.array([0, 1], np.uint32))
    n = _B * _D
    hi = np.zeros(n, np.uint32)
    lo = np.arange(n, dtype=np.uint32)
    out = []
    for kk1, kk2 in ((b1[0], b2[0]), (b1[1], b2[1])):
        r1, r2 = _tf2x32(kk1, kk2, hi, lo)
        bits = (r1 ^ r2).astype(np.uint32)
        float_bits = (bits >> np.uint32(9)) | np.uint32(0x3F800000)
        floats = float_bits.view(np.float32) - np.float32(1.0)
        minval = np.nextafter(np.float32(-1.0), np.float32(0.0), dtype=np.float32)
        u = np.maximum(minval, (floats * (np.float32(1.0) - minval)
                                + minval).astype(np.float32))
        out.append((np.float32(math.sqrt(2)) * _erfinv_f32(u))
                   .astype(np.float32).reshape(_B, _D))
    return out[0], out[1]


def _gather_one_body(ids, tbl, rows_o, idx_v, g_v, sem):
    wid = lax.axis_index("s") * _NC + lax.axis_index("c")
    base0 = wid * _B_PER_W

    def chunk_fn(ci, carry):
        base = base0 + ci * _CHUNK
        pltpu.sync_copy(ids.at[pl.ds(base, _CHUNK)], idx_v)
        pltpu.async_copy(tbl.at[idx_v], g_v, sem).wait()
        pltpu.sync_copy(g_v, rows_o.at[pl.ds(base, _CHUNK)])
        return carry

    lax.fori_loop(0, _B_PER_W // _CHUNK, chunk_fn, 0)


def _reparam_body(r_mu_s, r_lv_s, r_mu_a, r_lv_a, eps_s_h, eps_a_h,
                  lat_s_o, lat_a_o,
                  mu_s_v, lv_s_v, mu_a_v, lv_a_v, eps_s_v, eps_a_v, sem):
    wid = lax.axis_index("s") * _NC + lax.axis_index("c")
    base0 = wid * _B_PER_W

    def chunk_fn(ci, carry0):
        base = base0 + ci * _CHUNK
        src = pl.ds(base, _CHUNK)
        copies = [
            pltpu.async_copy(r_mu_s.at[src], mu_s_v, sem),
            pltpu.async_copy(r_lv_s.at[src], lv_s_v, sem),
            pltpu.async_copy(r_mu_a.at[src], mu_a_v, sem),
            pltpu.async_copy(r_lv_a.at[src], lv_a_v, sem),
            pltpu.async_copy(eps_s_h.at[src], eps_s_v, sem),
            pltpu.async_copy(eps_a_h.at[src], eps_a_v, sem),
        ]
        for cp in copies:
            cp.wait()

        def row_fn(r0, carry):
            for u in range(4):
                r = r0 * 4 + u
                for j in range(_D // _L):
                    sl = pl.ds(j * _L, _L)
                    eps_s_v[r, sl] = mu_s_v[r, sl] + eps_s_v[r, sl] * jnp.exp(
                        0.5 * lv_s_v[r, sl])
                    eps_a_v[r, sl] = mu_a_v[r, sl] + eps_a_v[r, sl] * jnp.exp(
                        0.5 * lv_a_v[r, sl])
            return carry

        lax.fori_loop(0, _CHUNK // 4, row_fn, 0)
        pltpu.sync_copy(eps_s_v, lat_s_o.at[src])
        pltpu.sync_copy(eps_a_v, lat_a_o.at[src])
        return carry0

    lax.fori_loop(0, _B_PER_W // _CHUNK, chunk_fn, 0)


@functools.lru_cache(maxsize=None)
def _build_gather_one():
    out = jax.ShapeDtypeStruct((_B, _D), jnp.float32)
    return pl.kernel(
        _gather_one_body,
        mesh=plsc.VectorSubcoreMesh(core_axis_name="c", subcore_axis_name="s"),
        compiler_params=pltpu.CompilerParams(use_tc_tiling_on_sc=False),
        out_type=out,
        scratch_types=[
            pltpu.VMEM((_CHUNK,), jnp.int32),
            pltpu.VMEM((_CHUNK, _D), jnp.float32),
            pltpu.SemaphoreType.DMA,
        ],
    )


@functools.lru_cache(maxsize=None)
def _build_reparam():
    out = jax.ShapeDtypeStruct((_B, _D), jnp.float32)
    return pl.kernel(
        _reparam_body,
        mesh=plsc.VectorSubcoreMesh(core_axis_name="c", subcore_axis_name="s"),
        compiler_params=pltpu.CompilerParams(use_tc_tiling_on_sc=True),
        out_type=[out] * 2,
        scratch_types=[
            pltpu.VMEM((_CHUNK, _D), jnp.float32),
            pltpu.VMEM((_CHUNK, _D), jnp.float32),
            pltpu.VMEM((_CHUNK, _D), jnp.float32),
            pltpu.VMEM((_CHUNK, _D), jnp.float32),
            pltpu.VMEM((_CHUNK, _D), jnp.float32),
            pltpu.VMEM((_CHUNK, _D), jnp.float32),
            pltpu.SemaphoreType.DMA,
        ],
    )


def kernel(instance_ids, weight_mu_shape, weight_logvar_shape,
           weight_mu_app, weight_logvar_app):
    ids = instance_ids.astype(jnp.int32)
    eps_s, eps_a = _eps_consts()
    gather_one = _build_gather_one()
    mu_s = gather_one(ids, weight_mu_shape)
    lv_s = gather_one(ids, weight_logvar_shape)
    mu_a = gather_one(ids, weight_mu_app)
    lv_a = gather_one(ids, weight_logvar_app)
    lat_s, lat_a = _build_reparam()(
        mu_s, lv_s, mu_a, lv_a, jnp.asarray(eps_s), jnp.asarray(eps_a))
    return (lat_s, lat_a, mu_s, lv_s, mu_a, lv_a)


# speedup vs baseline: 1.4985x; 1.4829x over previous
import functools
import math

import numpy as np
import jax
import jax.numpy as jnp
from jax import lax
from jax.experimental import pallas as pl
from jax.experimental.pallas import tpu as pltpu
from jax.experimental.pallas import tpu_sc as plsc

_B = 16384
_D = 64

_info = plsc.get_sparse_core_info()
_NC, _NS, _L = _info.num_cores, _info.num_subcores, _info.num_lanes
_NW = _NC * _NS
_B_PER_W = _B // _NW
_CHUNK = 128


def _tf2x32(k1, k2, x0, x1):
    ks0 = np.uint32(k1)
    ks1 = np.uint32(k2)
    ks2 = np.uint32(ks0 ^ ks1 ^ np.uint32(0x1BD11BDA))
    x0 = x0.astype(np.uint32)
    x1 = x1.astype(np.uint32)
    rot0 = (13, 15, 26, 6)
    rot1 = (17, 29, 16, 24)

    def rnd(x0, x1, rots):
        for r in rots:
            x0 = (x0 + x1).astype(np.uint32)
            x1 = ((x1 << np.uint32(r)) | (x1 >> np.uint32(32 - r))).astype(np.uint32)
            x1 = x0 ^ x1
        return x0, x1

    x0 = (x0 + ks0).astype(np.uint32)
    x1 = (x1 + ks1).astype(np.uint32)
    x0, x1 = rnd(x0, x1, rot0)
    x0 = (x0 + ks1).astype(np.uint32); x1 = (x1 + ks2 + np.uint32(1)).astype(np.uint32)
    x0, x1 = rnd(x0, x1, rot1)
    x0 = (x0 + ks2).astype(np.uint32); x1 = (x1 + ks0 + np.uint32(2)).astype(np.uint32)
    x0, x1 = rnd(x0, x1, rot0)
    x0 = (x0 + ks0).astype(np.uint32); x1 = (x1 + ks1 + np.uint32(3)).astype(np.uint32)
    x0, x1 = rnd(x0, x1, rot1)
    x0 = (x0 + ks1).astype(np.uint32); x1 = (x1 + ks2 + np.uint32(4)).astype(np.uint32)
    x0, x1 = rnd(x0, x1, rot0)
    x0 = (x0 + ks2).astype(np.uint32); x1 = (x1 + ks0 + np.uint32(5)).astype(np.uint32)
    return x0, x1


def _erfinv_f32(x):
    x = x.astype(np.float32)
    w = -np.log1p((-x * x).astype(np.float32)).astype(np.float32)
    small = w < np.float32(5.0)
    ws = (w - np.float32(2.5)).astype(np.float32)
    wl = (np.sqrt(w.astype(np.float32)) - np.float32(3.0)).astype(np.float32)
    cs = (2.81022636e-08, 3.43273939e-07, -3.5233877e-06, -4.39150654e-06,
          0.00021858087, -0.00125372503, -0.00417768164, 0.246640727, 1.50140941)
    cl = (-0.000200214257, 0.000100950558, 0.00134934322, -0.00367342844,
          0.00573950773, -0.0076224613, 0.00943887047, 1.00167406, 2.83297682)
    ps = np.float32(cs[0])
    for c in cs[1:]:
        ps = (np.float32(c) + ps * ws).astype(np.float32)
    plg = np.float32(cl[0])
    for c in cl[1:]:
        plg = (np.float32(c) + plg * wl).astype(np.float32)
    return (np.where(small, ps, plg) * x).astype(np.float32)


@functools.lru_cache(maxsize=None)
def _eps_consts():
    k1, k2 = np.uint32(0), np.uint32(42)
    b1, b2 = _tf2x32(k1, k2, np.array([0, 0], np.uint32),
                     np.array([0, 1], np.uint32))
    n = _B * _D
    hi = np.zeros(n, np.uint32)
    lo = np.arange(n, dtype=np.uint32)
    out = []
    for kk1, kk2 in ((b1[0], b2[0]), (b1[1], b2[1])):
        r1, r2 = _tf2x32(kk1, kk2, hi, lo)
        bits = (r1 ^ r2).astype(np.uint32)
        float_bits = (bits >> np.uint32(9)) | np.uint32(0x3F800000)
        floats = float_bits.view(np.float32) - np.float32(1.0)
        minval = np.nextafter(np.float32(-1.0), np.float32(0.0), dtype=np.float32)
        u = np.maximum(minval, (floats * (np.float32(1.0) - minval)
                                + minval).astype(np.float32))
        out.append((np.float32(math.sqrt(2)) * _erfinv_f32(u))
                   .astype(np.float32).reshape(_B, _D))
    return out[0], out[1]


def _sc_body(ids, t_mu_s, t_lv_s, t_mu_a, t_lv_a, eps_s_h, eps_a_h,
             lat_s_o, lat_a_o, mu_s_o, lv_s_o, mu_a_o, lv_a_o,
             idx_v, mu_s_v, lv_s_v, mu_a_v, lv_a_v, eps_s_v, eps_a_v,
             sem0, sem1, sem2, sem3, esem):
    wid = lax.axis_index("s") * _NC + lax.axis_index("c")
    base0 = wid * _B_PER_W
    for ci in range(_B_PER_W // _CHUNK):
        base = base0 + ci * _CHUNK
        pltpu.sync_copy(ids.at[pl.ds(base, _CHUNK)], idx_v)
        e1 = pltpu.async_copy(eps_s_h.at[pl.ds(base, _CHUNK)], eps_s_v, esem)
        e2 = pltpu.async_copy(eps_a_h.at[pl.ds(base, _CHUNK)], eps_a_v, esem)

        def fire(g, carry):
            idv = idx_v[pl.ds(g * _L, _L)]
            for k in range(_L):
                rid = idv[k]
                src = pl.ds(rid, 1)
                dst = pl.ds(g * _L + k, 1)
                pltpu.async_copy(t_mu_s.at[src], mu_s_v.at[dst], sem0)
                pltpu.async_copy(t_lv_s.at[src], lv_s_v.at[dst], sem1)
                pltpu.async_copy(t_mu_a.at[src], mu_a_v.at[dst], sem2)
                pltpu.async_copy(t_lv_a.at[src], lv_a_v.at[dst], sem3)
            return carry

        lax.fori_loop(0, _CHUNK // _L, fire, 0)
        for buf, sem in ((mu_s_v, sem0), (lv_s_v, sem1),
                         (mu_a_v, sem2), (lv_a_v, sem3)):
            pltpu.make_async_copy(t_mu_s.at[pl.ds(0, _CHUNK)], buf, sem).wait()
        e1.wait()
        e2.wait()

        def row_fn(r0, carry):
            for u in range(4):
                r = r0 * 4 + u
                for j in range(_D // _L):
                    sl = pl.ds(j * _L, _L)
                    eps_s_v[r, sl] = mu_s_v[r, sl] + eps_s_v[r, sl] * jnp.exp(
                        0.5 * lv_s_v[r, sl])
                    eps_a_v[r, sl] = mu_a_v[r, sl] + eps_a_v[r, sl] * jnp.exp(
                        0.5 * lv_a_v[r, sl])
            return carry

        lax.fori_loop(0, _CHUNK // 4, row_fn, 0)

        dst = pl.ds(base, _CHUNK)
        pltpu.sync_copy(eps_s_v, lat_s_o.at[dst])
        pltpu.sync_copy(eps_a_v, lat_a_o.at[dst])
        pltpu.sync_copy(mu_s_v, mu_s_o.at[dst])
        pltpu.sync_copy(lv_s_v, lv_s_o.at[dst])
        pltpu.sync_copy(mu_a_v, mu_a_o.at[dst])
        pltpu.sync_copy(lv_a_v, lv_a_o.at[dst])


@functools.lru_cache(maxsize=None)
def _build_kernel():
    out = jax.ShapeDtypeStruct((_B, _D), jnp.float32)
    return pl.kernel(
        _sc_body,
        mesh=plsc.VectorSubcoreMesh(core_axis_name="c", subcore_axis_name="s"),
        compiler_params=pltpu.CompilerParams(use_tc_tiling_on_sc=True),
        out_type=[out] * 6,
        scratch_types=[
            pltpu.VMEM((_CHUNK,), jnp.int32),
            pltpu.VMEM((_CHUNK, _D), jnp.float32),
            pltpu.VMEM((_CHUNK, _D), jnp.float32),
            pltpu.VMEM((_CHUNK, _D), jnp.float32),
            pltpu.VMEM((_CHUNK, _D), jnp.float32),
            pltpu.VMEM((_CHUNK, _D), jnp.float32),
            pltpu.VMEM((_CHUNK, _D), jnp.float32),
            pltpu.SemaphoreType.DMA,
            pltpu.SemaphoreType.DMA,
            pltpu.SemaphoreType.DMA,
            pltpu.SemaphoreType.DMA,
            pltpu.SemaphoreType.DMA,
        ],
    )


def kernel(instance_ids, weight_mu_shape, weight_logvar_shape,
           weight_mu_app, weight_logvar_app):
    ids = instance_ids.astype(jnp.int32)
    eps_s, eps_a = _eps_consts()
    lat_s, lat_a, mu_s, lv_s, mu_a, lv_a = _build_kernel()(
        ids, weight_mu_shape, weight_logvar_shape,
        weight_mu_app, weight_logvar_app,
        jnp.asarray(eps_s), jnp.asarray(eps_a))
    return (lat_s, lat_a, mu_s, lv_s, mu_a, lv_a)
